# initial kernel scaffold (unmeasured)
import jax
import jax.numpy as jnp
from jax import lax
from jax.experimental import pallas as pl
from jax.experimental.pallas import tpu as pltpu


def kernel(
    x,
):
    def body(*refs):
        pass

    out_shape = jax.ShapeDtypeStruct(..., jnp.float32)
    return pl.pallas_call(body, out_shape=out_shape)(...)


# baseline (device time: 4377722 ns/iter reference)
import jax
import jax.numpy as jnp
from jax import lax
from jax.experimental import pallas as pl
from jax.experimental.pallas import tpu as pltpu

N_DEV = 4


def kernel(x):
    m_per, n = x.shape

    def body(x_ref, out_ref, copy_sem, send_sems, recv_sems):
        my_pos = lax.axis_index("i")
        right = lax.rem(my_pos + 1, N_DEV)

        cp = pltpu.make_async_copy(
            x_ref, out_ref.at[pl.ds(my_pos * m_per, m_per), :], copy_sem
        )
        cp.start()
        cp.wait()

        for h in range(N_DEV - 1):
            origin = lax.rem(my_pos - h + N_DEV, N_DEV)
            sl = pl.ds(origin * m_per, m_per)
            rdma = pltpu.make_async_remote_copy(
                src_ref=out_ref.at[sl, :],
                dst_ref=out_ref.at[sl, :],
                send_sem=send_sems.at[h],
                recv_sem=recv_sems.at[h],
                device_id=(right,),
                device_id_type=pl.DeviceIdType.MESH,
            )
            rdma.start()
            rdma.wait()

    return pl.pallas_call(
        body,
        out_shape=jax.ShapeDtypeStruct((N_DEV * m_per, n), x.dtype),
        in_specs=[pl.BlockSpec(memory_space=pl.ANY)],
        out_specs=pl.BlockSpec(memory_space=pl.ANY),
        scratch_shapes=[
            pltpu.SemaphoreType.DMA,
            pltpu.SemaphoreType.DMA((N_DEV - 1,)),
            pltpu.SemaphoreType.DMA((N_DEV - 1,)),
        ],
    )(x)


# device time: 3298790 ns/iter; 1.3271x vs baseline; 1.3271x over previous
import jax
import jax.numpy as jnp
from jax import lax
from jax.experimental import pallas as pl
from jax.experimental.pallas import tpu as pltpu

N_DEV = 4


def kernel(x):
    m_per, n = x.shape
    m_half = m_per // 2

    def body(
        x_ref,
        out_ref,
        copy_sem,
        send_sems_r,
        recv_sems_r,
        send_sems_l,
        recv_sems_l,
    ):
        my_pos = lax.axis_index("i")
        right = lax.rem(my_pos + 1, N_DEV)
        left = lax.rem(my_pos + N_DEV - 1, N_DEV)

        cp = pltpu.make_async_copy(
            x_ref, out_ref.at[pl.ds(my_pos * m_per, m_per), :], copy_sem
        )
        cp.start()
        cp.wait()

        for h in range(N_DEV - 1):
            origin_r = lax.rem(my_pos - h + N_DEV, N_DEV)
            origin_l = lax.rem(my_pos + h, N_DEV)
            sl_r = pl.ds(origin_r * m_per, m_half)
            sl_l = pl.ds(origin_l * m_per + m_half, m_half)

            rdma_r = pltpu.make_async_remote_copy(
                src_ref=out_ref.at[sl_r, :],
                dst_ref=out_ref.at[sl_r, :],
                send_sem=send_sems_r.at[h],
                recv_sem=recv_sems_r.at[h],
                device_id=(right,),
                device_id_type=pl.DeviceIdType.MESH,
            )
            rdma_l = pltpu.make_async_remote_copy(
                src_ref=out_ref.at[sl_l, :],
                dst_ref=out_ref.at[sl_l, :],
                send_sem=send_sems_l.at[h],
                recv_sem=recv_sems_l.at[h],
                device_id=(left,),
                device_id_type=pl.DeviceIdType.MESH,
            )
            rdma_r.start()
            rdma_l.start()
            rdma_r.wait()
            rdma_l.wait()

    return pl.pallas_call(
        body,
        out_shape=jax.ShapeDtypeStruct((N_DEV * m_per, n), x.dtype),
        in_specs=[pl.BlockSpec(memory_space=pl.ANY)],
        out_specs=pl.BlockSpec(memory_space=pl.ANY),
        scratch_shapes=[
            pltpu.SemaphoreType.DMA,
            pltpu.SemaphoreType.DMA((N_DEV - 1,)),
            pltpu.SemaphoreType.DMA((N_DEV - 1,)),
            pltpu.SemaphoreType.DMA((N_DEV - 1,)),
            pltpu.SemaphoreType.DMA((N_DEV - 1,)),
        ],
    )(x)


# device time: 1260473 ns/iter; 3.4731x vs baseline; 2.6171x over previous
import jax
import jax.numpy as jnp
from jax import lax
from jax.experimental import pallas as pl
from jax.experimental.pallas import tpu as pltpu

N_DEV = 4
N_HOPS = N_DEV - 1
STAGE_CHUNKS = 8


def kernel(x):
    m_per, n = x.shape
    m_half = m_per // 2
    m_stage = m_per // STAGE_CHUNKS

    def body(
        x_ref,
        out_ref,
        stage_vmem,
        load_sems,
        store_sems,
        send_sems_r,
        recv_sems_r,
        send_sems_l,
        recv_sems_l,
    ):
        my_pos = lax.axis_index("i")
        right = lax.rem(my_pos + 1, N_DEV)
        left = lax.rem(my_pos + N_DEV - 1, N_DEV)
        my_row = my_pos * m_per

        def hop_rdmas(h):
            origin_r = lax.rem(my_pos - h + N_DEV, N_DEV)
            origin_l = lax.rem(my_pos + h, N_DEV)
            if h == 0:
                src_r = x_ref.at[pl.ds(0, m_half), :]
                src_l = x_ref.at[pl.ds(m_half, m_half), :]
            else:
                src_r = out_ref.at[pl.ds(origin_r * m_per, m_half), :]
                src_l = out_ref.at[pl.ds(origin_l * m_per + m_half, m_half), :]
            rdma_r = pltpu.make_async_remote_copy(
                src_ref=src_r,
                dst_ref=out_ref.at[pl.ds(origin_r * m_per, m_half), :],
                send_sem=send_sems_r.at[h],
                recv_sem=recv_sems_r.at[h],
                device_id=(right,),
                device_id_type=pl.DeviceIdType.MESH,
            )
            rdma_l = pltpu.make_async_remote_copy(
                src_ref=src_l,
                dst_ref=out_ref.at[pl.ds(origin_l * m_per + m_half, m_half), :],
                send_sem=send_sems_l.at[h],
                recv_sem=recv_sems_l.at[h],
                device_id=(left,),
                device_id_type=pl.DeviceIdType.MESH,
            )
            return rdma_r, rdma_l

        r0, l0 = hop_rdmas(0)
        r0.start()
        l0.start()

        def load(c, slot):
            return pltpu.make_async_copy(
                x_ref.at[pl.ds(c * m_stage, m_stage), :],
                stage_vmem.at[slot],
                load_sems.at[slot],
            )

        def store(c, slot):
            return pltpu.make_async_copy(
                stage_vmem.at[slot],
                out_ref.at[pl.ds(my_row + c * m_stage, m_stage), :],
                store_sems.at[slot],
            )

        load(0, 0).start()
        for c in range(STAGE_CHUNKS):
            slot = c % 2
            load(c, slot).wait()
            if c + 1 < STAGE_CHUNKS:
                load(c + 1, (c + 1) % 2).start()
            st = store(c, slot)
            st.start()
            st.wait()

        r0.wait()
        l0.wait()

        for h in range(1, N_HOPS):
            rdma_r, rdma_l = hop_rdmas(h)
            rdma_r.start()
            rdma_l.start()
            rdma_r.wait()
            rdma_l.wait()

    return pl.pallas_call(
        body,
        out_shape=jax.ShapeDtypeStruct((N_DEV * m_per, n), x.dtype),
        in_specs=[pl.BlockSpec(memory_space=pl.ANY)],
        out_specs=pl.BlockSpec(memory_space=pl.ANY),
        scratch_shapes=[
            pltpu.MemorySpace.VMEM((2, m_stage, n), x.dtype),
            pltpu.SemaphoreType.DMA((2,)),
            pltpu.SemaphoreType.DMA((2,)),
            pltpu.SemaphoreType.DMA((N_HOPS,)),
            pltpu.SemaphoreType.DMA((N_HOPS,)),
            pltpu.SemaphoreType.DMA((N_HOPS,)),
            pltpu.SemaphoreType.DMA((N_HOPS,)),
        ],
    )(x)
